# baseline (speedup 1.0000x reference)
import functools

import jax
import jax.numpy as jnp
from jax import lax
from jax.experimental import pallas as pl
from jax.experimental.pallas import tpu as pltpu
from jax.experimental.pallas import tpu_sc as plsc

NC, NS = 2, 16
NW = NC * NS
GW = 100
GPS = 4
CB0 = 2
NPH = 2


@functools.partial(jax.jit, static_argnames=("B0", "B1", "D"))
def _sc_gather(tablep, idx, B0, B1, D):
    b0_per_w = B0 // NW
    b_per_w = b0_per_w * B1
    n = b0_per_w // CB0
    n_p = n // NPH
    gr_p = n_p * GPS
    assert CB0 * B1 == GW * GPS and n_p >= 4 and n_p % 2 == 0
    idx3 = idx.reshape(NW, b_per_w // GW, GW)
    mesh = plsc.VectorSubcoreMesh(core_axis_name="c", subcore_axis_name="s")

    @functools.partial(
        pl.kernel,
        out_type=jax.ShapeDtypeStruct((B0, B1, 2 * D), jnp.float32),
        mesh=mesh,
        scratch_types=[
            pltpu.VMEM((gr_p, GW), jnp.int32),
            pltpu.VMEM((2, CB0, B1, 2 * D), jnp.float32),
            pltpu.SemaphoreType.DMA,
            pltpu.SemaphoreType.DMA,
            pltpu.SemaphoreType.DMA,
            pltpu.SemaphoreType.DMA,
        ],
        compiler_params=pltpu.CompilerParams(use_tc_tiling_on_sc=True),
    )
    def k(table_hbm, idx_hbm, out_hbm, idx_v, rows_v, g0, g1, w0, w1):
        wid = lax.axis_index("s") * NC + lax.axis_index("c")
        b0_base = wid * b0_per_w
        gsem = (g0, g1)
        wsem = (w0, w1)

        def phase(p):
            pltpu.sync_copy(idx_hbm.at[wid, pl.ds(p * gr_p, gr_p)], idx_v)
            base = p * n_p

            def fire_g(g, par):
                for j in range(GPS):
                    pltpu.async_copy(
                        table_hbm.at[idx_v.at[g * GPS + j]],
                        rows_v.at[par, (j * GW) // B1, pl.ds((j * GW) % B1, GW)],
                        gsem[par],
                    )

            def drain_g(par):
                pltpu.make_async_copy(
                    out_hbm.at[pl.ds(0, CB0)], rows_v.at[par], gsem[par]
                ).wait()

            def fire_w(g, par):
                pltpu.async_copy(
                    rows_v.at[par],
                    out_hbm.at[pl.ds(b0_base + (base + g) * CB0, CB0)],
                    wsem[par],
                )

            def drain_w(par):
                pltpu.make_async_copy(
                    rows_v.at[par], out_hbm.at[pl.ds(0, CB0)], wsem[par]
                ).wait()

            fire_g(0, 0)
            fire_g(1, 1)
            drain_g(0)
            fire_w(0, 0)

            @pl.loop(0, (n_p - 4) // 2)
            def superstep(s):
                godd = 2 * s + 1
                drain_w(0)
                fire_g(godd + 1, 0)
                drain_g(1)
                fire_w(godd, 1)
                drain_w(1)
                fire_g(godd + 2, 1)
                drain_g(0)
                fire_w(godd + 1, 0)

            drain_w(0)
            fire_g(n_p - 2, 0)
            drain_g(1)
            fire_w(n_p - 3, 1)
            drain_w(1)
            fire_g(n_p - 1, 1)
            drain_g(0)
            fire_w(n_p - 2, 0)
            drain_g(1)
            fire_w(n_p - 1, 1)
            drain_w(0)
            drain_w(1)

        for p in range(NPH):
            phase(p)

    return k(tablep, idx3)


def kernel(x, table):
    B0, B1 = x.shape
    V, D = table.shape
    xf = x.reshape(-1).astype(jnp.int32)
    tablep = jnp.pad(table, ((0, 0), (0, D)))
    out2 = _sc_gather(tablep, xf, B0, B1, D)
    return out2[:, :, :D]


# speedup vs baseline: 1.0021x; 1.0021x over previous
import functools

import jax
import jax.numpy as jnp
from jax import lax
from jax.experimental import pallas as pl
from jax.experimental.pallas import tpu as pltpu
from jax.experimental.pallas import tpu_sc as plsc

NC, NS = 2, 16
NW = NC * NS
GW = 100
GPS = 2
NBUF = 3


@functools.partial(jax.jit, static_argnames=("B0", "B1", "D"))
def _sc_gather(tablep, idx, B0, B1, D):
    b0_per_w = B0 // NW
    b_per_w = b0_per_w * B1
    n = b0_per_w
    assert B1 == GW * GPS and n >= 8 and (n - 3) % 3 == 2
    idx3 = idx.reshape(NW, b_per_w // GW, GW)
    mesh = plsc.VectorSubcoreMesh(core_axis_name="c", subcore_axis_name="s")

    @functools.partial(
        pl.kernel,
        out_type=jax.ShapeDtypeStruct((B0, B1, 2 * D), jnp.float32),
        mesh=mesh,
        scratch_types=[
            pltpu.VMEM((b_per_w // GW, GW), jnp.int32),
            pltpu.VMEM((NBUF, B1, 2 * D), jnp.float32),
            pltpu.SemaphoreType.DMA,
            pltpu.SemaphoreType.DMA,
            pltpu.SemaphoreType.DMA,
            pltpu.SemaphoreType.DMA,
            pltpu.SemaphoreType.DMA,
            pltpu.SemaphoreType.DMA,
        ],
        compiler_params=pltpu.CompilerParams(use_tc_tiling_on_sc=True),
    )
    def k(table_hbm, idx_hbm, out_hbm, idx_v, rows_v, g0, g1, g2, w0, w1, w2):
        wid = lax.axis_index("s") * NC + lax.axis_index("c")
        pltpu.sync_copy(idx_hbm.at[wid], idx_v)
        b0_base = wid * b0_per_w
        gsem = (g0, g1, g2)
        wsem = (w0, w1, w2)

        def fire_g(g, buf):
            for j in range(GPS):
                pltpu.async_copy(
                    table_hbm.at[idx_v.at[g * GPS + j]],
                    rows_v.at[buf, pl.ds(j * GW, GW)],
                    gsem[buf],
                )

        def drain_g(buf):
            pltpu.make_async_copy(
                table_hbm.at[pl.ds(0, B1)], rows_v.at[buf], gsem[buf]
            ).wait()

        def fire_w(g, buf):
            pltpu.async_copy(
                rows_v.at[buf],
                out_hbm.at[b0_base + g],
                wsem[buf],
            )

        def drain_w(buf):
            pltpu.make_async_copy(
                rows_v.at[buf], out_hbm.at[0], wsem[buf]
            ).wait()

        fire_g(0, 0)
        fire_g(1, 1)
        drain_g(0)
        fire_w(0, 0)
        fire_g(2, 2)
        drain_g(1)
        fire_w(1, 1)

        @pl.loop(0, (n - 5) // 3)
        def superstep(s):
            ga = 3 * s + 2
            drain_w(0)
            fire_g(ga + 1, 0)
            drain_g(2)
            fire_w(ga, 2)
            drain_w(1)
            fire_g(ga + 2, 1)
            drain_g(0)
            fire_w(ga + 1, 0)
            drain_w(2)
            fire_g(ga + 3, 2)
            drain_g(1)
            fire_w(ga + 2, 1)

        drain_w(0)
        fire_g(n - 2, 0)
        drain_g(2)
        fire_w(n - 3, 2)
        drain_w(1)
        fire_g(n - 1, 1)
        drain_g(0)
        fire_w(n - 2, 0)
        drain_w(2)
        drain_g(1)
        fire_w(n - 1, 1)
        drain_w(0)
        drain_w(1)

    return k(tablep, idx3)


def kernel(x, table):
    B0, B1 = x.shape
    V, D = table.shape
    xf = x.reshape(-1).astype(jnp.int32)
    tablep = jnp.pad(table, ((0, 0), (0, D)))
    out2 = _sc_gather(tablep, xf, B0, B1, D)
    return out2[:, :, :D]
